# baseline (speedup 1.0000x reference)
import functools

import jax
import jax.numpy as jnp
from jax import lax
from jax.experimental import pallas as pl
from jax.experimental.pallas import tpu as pltpu
from jax.experimental.pallas import tpu_sc as plsc

N_NODES = 10000
N_EDGES = 160000
D_FEAT = 256

NC, NS, L = 2, 16, 16
NW = NC * NS
E_PAD = 163840
EW = E_PAD // NW
C = 64
NCHUNK = EW // C
G = C // L

_mesh = plsc.VectorSubcoreMesh(core_axis_name="c", subcore_axis_name="s")


@functools.partial(
    pl.kernel,
    out_type=jax.ShapeDtypeStruct((NW, NCHUNK, C), jnp.float32),
    mesh=_mesh,
    scratch_types=[
        pltpu.VMEM((NCHUNK, C), jnp.int32),
        pltpu.VMEM((NCHUNK, C), jnp.int32),
        pltpu.VMEM((NCHUNK, C), jnp.float32),
        pltpu.VMEM((2, C, D_FEAT), jnp.float32),
        pltpu.VMEM((2, C, D_FEAT), jnp.float32),
        pltpu.SemaphoreType.DMA,
        pltpu.SemaphoreType.DMA,
    ],
    compiler_params=pltpu.CompilerParams(
        use_tc_tiling_on_sc=False, needs_layout_passes=False
    ),
)
def _sc_scores(h_hbm, u_hbm, v_hbm, out_hbm, u_v, v_v, sc_v, sr, dr, s0, s1):
    wid = lax.axis_index("s") * NC + lax.axis_index("c")
    sems = (s0, s1)

    pltpu.sync_copy(u_hbm.at[wid], u_v)
    pltpu.sync_copy(v_hbm.at[wid], v_v)

    def issue(k, b):
        pltpu.async_copy(h_hbm.at[u_v.at[k]], sr.at[b], sems[b])
        pltpu.async_copy(h_hbm.at[v_v.at[k]], dr.at[b], sems[b])

    def drain(b):
        pltpu.make_async_copy(h_hbm.at[u_v.at[0]], sr.at[b], sems[b]).wait()
        pltpu.make_async_copy(h_hbm.at[v_v.at[0]], dr.at[b], sems[b]).wait()

    iota = lax.iota(jnp.int32, L)

    def compute(k, b):
        for g in range(G):
            eidx = iota + (g * L)

            def dbody(d, acc):
                cold = jnp.full((L,), 0, jnp.int32) + d
                a = plsc.load_gather(sr.at[b], [eidx, cold])
                c = plsc.load_gather(dr.at[b], [eidx, cold])
                return acc + a * c

            acc = lax.fori_loop(
                0, D_FEAT, dbody, jnp.zeros((L,), jnp.float32), unroll=8
            )
            sc_v[k, pl.ds(g * L, L)] = acc

    issue(0, 0)

    def outer(k2, carry):
        for b in range(2):
            k = k2 * 2 + b

            @pl.when(k + 1 < NCHUNK)
            def _():
                issue(k + 1, (b + 1) % 2)

            drain(b)
            compute(k, b)
        return carry

    lax.fori_loop(0, NCHUNK // 2, outer, 0)

    pltpu.sync_copy(sc_v, out_hbm.at[wid])


def kernel(h, edge_index):
    ei = edge_index.astype(jnp.int32)
    pad = jnp.zeros((E_PAD - N_EDGES,), jnp.int32)
    u = jnp.concatenate([ei[0], pad]).reshape(NW, NCHUNK, C)
    v = jnp.concatenate([ei[1], pad]).reshape(NW, NCHUNK, C)
    scores = _sc_scores(h, u, v)
    return scores.reshape(-1)[:N_EDGES]


# speedup vs baseline: 1.7038x; 1.7038x over previous
import functools

import jax
import jax.numpy as jnp
from jax import lax
from jax.experimental import pallas as pl
from jax.experimental.pallas import tpu as pltpu
from jax.experimental.pallas import tpu_sc as plsc

N_NODES = 10000
N_EDGES = 160000
D_FEAT = 256
DW = D_FEAT // 2

NC, NS, L = 2, 16, 16
NW = NC * NS
E_PAD = 163840
EW = E_PAD // NW
C = 128
NCHUNK = EW // C
G = C // L

_mesh = plsc.VectorSubcoreMesh(core_axis_name="c", subcore_axis_name="s")


@functools.partial(
    pl.kernel,
    out_type=jax.ShapeDtypeStruct((NW, NCHUNK, C), jnp.float32),
    mesh=_mesh,
    scratch_types=[
        pltpu.VMEM((NCHUNK, C), jnp.int32),
        pltpu.VMEM((NCHUNK, C), jnp.int32),
        pltpu.VMEM((NCHUNK, C), jnp.float32),
        pltpu.VMEM((2, C, DW), jnp.int32),
        pltpu.VMEM((2, C, DW), jnp.int32),
        pltpu.SemaphoreType.DMA,
        pltpu.SemaphoreType.DMA,
    ],
    compiler_params=pltpu.CompilerParams(
        use_tc_tiling_on_sc=False, needs_layout_passes=False
    ),
)
def _sc_scores(h_hbm, u_hbm, v_hbm, out_hbm, u_v, v_v, sc_v, sr, dr, s0, s1):
    wid = lax.axis_index("s") * NC + lax.axis_index("c")
    sems = (s0, s1)

    pltpu.sync_copy(u_hbm.at[wid], u_v)
    pltpu.sync_copy(v_hbm.at[wid], v_v)

    def issue(k, b):
        pltpu.async_copy(h_hbm.at[u_v.at[k]], sr.at[b], sems[b])
        pltpu.async_copy(h_hbm.at[v_v.at[k]], dr.at[b], sems[b])

    def drain(b):
        pltpu.make_async_copy(h_hbm.at[u_v.at[0]], sr.at[b], sems[b]).wait()
        pltpu.make_async_copy(h_hbm.at[v_v.at[0]], dr.at[b], sems[b]).wait()

    iota = lax.iota(jnp.int32, L)

    def compute(k, b):
        for g in range(G):
            eidx = iota + (g * L)
            z = jnp.zeros((L,), jnp.float32)

            @plsc.parallel_loop(0, DW, unroll=8, carry=(z, z))
            def accs(d, carry):
                acc0, acc1 = carry
                cold = jnp.full((L,), 0, jnp.int32) + d
                a = plsc.load_gather(sr.at[b], [eidx, cold])
                c = plsc.load_gather(dr.at[b], [eidx, cold])
                p = plsc.bitcast(a, jnp.bfloat16) * plsc.bitcast(c, jnp.bfloat16)
                p0, p1 = plsc.unpack(p, format=plsc.PackFormat.INTERLEAVED)
                return acc0 + p0, acc1 + p1

            acc0, acc1 = accs
            sc_v[k, pl.ds(g * L, L)] = acc0 + acc1

    issue(0, 0)

    def outer(k2, carry):
        for b in range(2):
            k = k2 * 2 + b

            @pl.when(k + 1 < NCHUNK)
            def _():
                issue(k + 1, (b + 1) % 2)

            drain(b)
            compute(k, b)
        return carry

    lax.fori_loop(0, NCHUNK // 2, outer, 0)

    pltpu.sync_copy(sc_v, out_hbm.at[wid])


def kernel(h, edge_index):
    ei = edge_index.astype(jnp.int32)
    h_pk = lax.bitcast_convert_type(
        h.astype(jnp.bfloat16).reshape(N_NODES, DW, 2), jnp.int32
    )
    pad = jnp.zeros((E_PAD - N_EDGES,), jnp.int32)
    u = jnp.concatenate([ei[0], pad]).reshape(NW, NCHUNK, C)
    v = jnp.concatenate([ei[1], pad]).reshape(NW, NCHUNK, C)
    scores = _sc_scores(h_pk, u, v)
    return scores.reshape(-1)[:N_EDGES]
